# Initial kernel scaffold; baseline (speedup 1.0000x reference)
import jax
import jax.numpy as jnp
from jax.experimental import pallas as pl


def kernel(x, seg, tok_embed, pos_embed, seg_embed, gamma, beta):
    raise NotImplementedError("write your pallas kernel here")


# speedup vs baseline: 1.6482x; 1.6482x over previous
import functools

import jax
import jax.numpy as jnp
from jax import lax
from jax.experimental import pallas as pl
from jax.experimental.pallas import tpu as pltpu
from jax.experimental.pallas import tpu_sc as plsc

LANES = 16
EPS = 1e-5


def _rsqrt16(v):
    i = lax.bitcast_convert_type(v, jnp.int32)
    i = jnp.int32(0x5F3759DF) - lax.shift_right_logical(i, 1)
    y = lax.bitcast_convert_type(i, jnp.float32)
    half = v * jnp.float32(0.5)
    for _ in range(3):
        y = y * (jnp.float32(1.5) - half * y * y)
    return y


def _make_sc_call(N, D, V, n_combo):
    info = plsc.get_sparse_core_info()
    NC, NS = info.num_cores, info.num_subcores
    NW = NC * NS
    assert N % NW == 0
    rows_per_worker = N // NW
    R = 64
    assert rows_per_worker % R == 0
    n_chunks = rows_per_worker // R
    n_slices = D // LANES

    mesh = plsc.VectorSubcoreMesh(core_axis_name="c", subcore_axis_name="s")

    @functools.partial(
        pl.kernel,
        mesh=mesh,
        compiler_params=pltpu.CompilerParams(needs_layout_passes=False),
        out_type=jax.ShapeDtypeStruct((N, D), jnp.float32),
        scratch_types=[
            pltpu.VMEM((R,), jnp.int32),
            pltpu.VMEM((R,), jnp.int32),
            pltpu.VMEM((R, D), jnp.float32),
            pltpu.VMEM((R, D), jnp.float32),
            pltpu.VMEM((D,), jnp.float32),
            pltpu.VMEM((D,), jnp.float32),
            pltpu.SemaphoreType.DMA,
            pltpu.SemaphoreType.DMA,
        ],
    )
    def sc_call(xf_h, cidx_h, tok_h, combo_h, gamma_h, beta_h, out_h,
                idx_v, cidx_v, rows_v, cbuf_v, g_v, b_v, sem1, sem2):
        wid = lax.axis_index("s") * NC + lax.axis_index("c")
        pltpu.sync_copy(gamma_h, g_v)
        pltpu.sync_copy(beta_h, b_v)

        def chunk_body(c, carry):
            base = wid * rows_per_worker + c * R
            pltpu.sync_copy(xf_h.at[pl.ds(base, R)], idx_v)
            pltpu.sync_copy(cidx_h.at[pl.ds(base, R)], cidx_v)
            cp1 = pltpu.async_copy(tok_h.at[idx_v], rows_v, sem1)
            cp2 = pltpu.async_copy(combo_h.at[cidx_v], cbuf_v, sem2)
            cp1.wait()
            cp2.wait()

            def row_body(r, rcarry):
                s1 = jnp.zeros((LANES,), jnp.float32)
                s2 = jnp.zeros((LANES,), jnp.float32)
                for j in range(n_slices):
                    off = j * LANES
                    t = rows_v[r, pl.ds(off, LANES)] + cbuf_v[r, pl.ds(off, LANES)]
                    rows_v[r, pl.ds(off, LANES)] = t
                    s1 = s1 + t
                    s2 = s2 + t * t
                tot = jnp.sum(s1)
                totsq = jnp.sum(s2)
                mean = tot * jnp.float32(1.0 / D)
                var = totsq * jnp.float32(1.0 / D) - mean * mean
                rstd = _rsqrt16(jnp.full((LANES,), var + jnp.float32(EPS),
                                         jnp.float32))
                meanv = jnp.full((LANES,), mean, jnp.float32)
                for j in range(n_slices):
                    off = j * LANES
                    t = rows_v[r, pl.ds(off, LANES)]
                    y = (t - meanv) * rstd * g_v[pl.ds(off, LANES)] \
                        + b_v[pl.ds(off, LANES)]
                    rows_v[r, pl.ds(off, LANES)] = y
                return rcarry

            lax.fori_loop(0, R, row_body, 0)
            pltpu.sync_copy(rows_v, out_h.at[pl.ds(base, R)])
            return carry

        lax.fori_loop(0, n_chunks, chunk_body, 0)

    return sc_call


def kernel(x, seg, tok_embed, pos_embed, seg_embed, gamma, beta):
    B, L = x.shape
    V, D = tok_embed.shape
    n_pos = pos_embed.shape[0]
    n_seg = seg_embed.shape[0]
    N = B * L

    xf = x.reshape(N).astype(jnp.int32)
    pos = jnp.broadcast_to(jnp.arange(L, dtype=jnp.int32)[None, :], (B, L))
    cidx = (pos * n_seg + seg.astype(jnp.int32)).reshape(N)
    combo = (pos_embed[:n_pos, None, :] + seg_embed[None, :, :]).reshape(
        n_pos * n_seg, D)

    sc_call = _make_sc_call(N, D, V, n_pos * n_seg)
    out = sc_call(xf, cidx, tok_embed, combo, gamma, beta)
    return out.reshape(B, L, D)
